# baseline (speedup 1.0000x reference)
import jax
import jax.numpy as jnp
from jax.experimental import pallas as pl
from jax.experimental.pallas import tpu as pltpu

_DIM = 2048
_NCLS = 512
_NQ = 1024
_NLEV = 8
_CHUNK = 1024
_NCHUNK = _DIM // _CHUNK
_DTOT = _DIM * 8
_NCH = 7
_KC = _NCH * _CHUNK
_MA = _NQ + 8


def _body(q_ref, w_ref, out_ref,
          qvm, wvm, qsq, wsq, wmn, wmx, tq, tw, enc_q, enc_w, acc, sq):
    i = pl.program_id(0)

    @pl.when(i == 0)
    def _init0():
        qsq[:] = jnp.zeros_like(qsq)
        wsq[:] = jnp.zeros_like(wsq)
        wmn[:] = jnp.full_like(wmn, jnp.inf)
        wmx[:] = jnp.full_like(wmx, -jnp.inf)

    @pl.when(i < _NCHUNK)
    def _phase0():
        q = q_ref[:]
        w = w_ref[:]
        qvm[i] = q.astype(jnp.bfloat16)
        wvm[i] = w.astype(jnp.bfloat16)
        qsq[:] += q * q
        wsq[:] += w * w
        wmn[:] = jnp.minimum(wmn[:], w)
        wmx[:] = jnp.maximum(wmx[:], w)

    @pl.when(i == _NCHUNK - 1)
    def _mk_thresholds():
        qnorm = jnp.sqrt(jnp.sum(qsq[:], axis=1, keepdims=True))
        wnorm = jnp.sqrt(jnp.sum(wsq[:], axis=1, keepdims=True))
        gmin = jnp.min(jnp.min(wmn[:], axis=1, keepdims=True) / wnorm)
        gmax = jnp.max(jnp.max(wmx[:], axis=1, keepdims=True) / wnorm)
        step = (gmax - gmin) / (_NLEV - 1)
        k = jax.lax.broadcasted_iota(jnp.int32, (1, 8), 1).astype(jnp.float32)
        mids = gmin + (k + 0.5) * step
        tq[:] = (mids * qnorm).astype(jnp.bfloat16)
        tw[:] = (mids * wnorm).astype(jnp.bfloat16)

    @pl.when(i == _NCHUNK)
    def _init1():
        acc[:] = jnp.zeros_like(acc)
        sq[:] = jnp.zeros_like(sq)
        row08 = jax.lax.broadcasted_iota(jnp.int32, (8, _KC), 0) == 0
        enc_q[_NQ:_MA, :] = jnp.where(row08, 1.0, 0.0).astype(jnp.bfloat16)

    @pl.when(i >= _NCHUNK)
    def _phase1():
        j = i - _NCHUNK
        q = qvm[j]
        w = wvm[j]
        one_q = jnp.ones_like(q)
        zero_q = jnp.zeros_like(q)
        one_w = jnp.ones_like(w)
        two_w = one_w + one_w
        zero_w = jnp.zeros_like(w)
        ones_q = zero_q
        for k in range(_NCH):
            cq = jnp.where(q > tq[:, k:k + 1], one_q, zero_q)
            cw = jnp.where(w > tw[:, k:k + 1],
                           two_w if k == 3 else one_w, zero_w)
            ones_q = ones_q + (cq + cq if k == 3 else cq)
            enc_q[0:_NQ, k * _CHUNK:(k + 1) * _CHUNK] = cq
            enc_w[0:_NCLS, k * _CHUNK:(k + 1) * _CHUNK] = cw
        sq[:] += jnp.sum(ones_q.astype(jnp.float32), axis=1, keepdims=True)
        acc[:] += jax.lax.dot_general(
            enc_q[:], enc_w[:],
            dimension_numbers=(((1,), (1,)), ((), ())),
            preferred_element_type=jnp.float32,
        )

    @pl.when(i == 2 * _NCHUNK - 1)
    def _fin():
        sw = acc[_NQ:_NQ + 1, 0:_NCLS]
        out_ref[:] = (_DTOT + 2.0 * acc[0:_NQ, 0:_NCLS]) - sq[:] - sw


def kernel(query, weight):
    return pl.pallas_call(
        _body,
        grid=(2 * _NCHUNK,),
        in_specs=[
            pl.BlockSpec((_NQ, _CHUNK),
                         lambda i: (0, jnp.minimum(i, _NCHUNK - 1))),
            pl.BlockSpec((_NCLS, _CHUNK),
                         lambda i: (0, jnp.minimum(i, _NCHUNK - 1))),
        ],
        out_specs=pl.BlockSpec((_NQ, _NCLS), lambda i: (0, 0)),
        out_shape=jax.ShapeDtypeStruct((_NQ, _NCLS), jnp.float32),
        scratch_shapes=[
            pltpu.VMEM((_NCHUNK, _NQ, _CHUNK), jnp.bfloat16),
            pltpu.VMEM((_NCHUNK, _NCLS, _CHUNK), jnp.bfloat16),
            pltpu.VMEM((_NQ, _CHUNK), jnp.float32),
            pltpu.VMEM((_NCLS, _CHUNK), jnp.float32),
            pltpu.VMEM((_NCLS, _CHUNK), jnp.float32),
            pltpu.VMEM((_NCLS, _CHUNK), jnp.float32),
            pltpu.VMEM((_NQ, 8), jnp.bfloat16),
            pltpu.VMEM((_NCLS, 8), jnp.bfloat16),
            pltpu.VMEM((_MA, _KC), jnp.bfloat16),
            pltpu.VMEM((_NCLS, _KC), jnp.bfloat16),
            pltpu.VMEM((_MA, _NCLS), jnp.float32),
            pltpu.VMEM((_NQ, 1), jnp.float32),
        ],
    )(query, weight)


# speedup vs baseline: 1.0235x; 1.0235x over previous
import jax
import jax.numpy as jnp
from jax.experimental import pallas as pl
from jax.experimental.pallas import tpu as pltpu

_DIM = 2048
_NCLS = 512
_NQ = 1024
_NLEV = 8
_CHUNK = 512
_NCHUNK = _DIM // _CHUNK
_DTOT = _DIM * 8
_NCH = 7
_KC = _NCH * _CHUNK
_MA = _NQ + 8


def _body(q_ref, w_ref, out_ref,
          qvm, wvm, qsq, wsq, wmn, wmx, tq, tw, enc_q, enc_w, acc, sq):
    i = pl.program_id(0)

    @pl.when(i == 0)
    def _init0():
        qsq[:] = jnp.zeros_like(qsq)
        wsq[:] = jnp.zeros_like(wsq)
        wmn[:] = jnp.full_like(wmn, jnp.inf)
        wmx[:] = jnp.full_like(wmx, -jnp.inf)

    @pl.when(i < _NCHUNK)
    def _phase0():
        q = q_ref[:]
        w = w_ref[:]
        qvm[i] = q.astype(jnp.bfloat16)
        wvm[i] = w.astype(jnp.bfloat16)
        qsq[:] += q * q
        wsq[:] += w * w
        wmn[:] = jnp.minimum(wmn[:], w)
        wmx[:] = jnp.maximum(wmx[:], w)

    @pl.when(i == _NCHUNK - 1)
    def _mk_thresholds():
        qnorm = jnp.sqrt(jnp.sum(qsq[:], axis=1, keepdims=True))
        wnorm = jnp.sqrt(jnp.sum(wsq[:], axis=1, keepdims=True))
        gmin = jnp.min(jnp.min(wmn[:], axis=1, keepdims=True) / wnorm)
        gmax = jnp.max(jnp.max(wmx[:], axis=1, keepdims=True) / wnorm)
        step = (gmax - gmin) / (_NLEV - 1)
        k = jax.lax.broadcasted_iota(jnp.int32, (1, 8), 1).astype(jnp.float32)
        mids = gmin + (k + 0.5) * step
        tq[:] = (mids * qnorm).astype(jnp.bfloat16)
        tw[:] = (mids * wnorm).astype(jnp.bfloat16)

    @pl.when(i == _NCHUNK)
    def _init1():
        acc[:] = jnp.zeros_like(acc)
        sq[:] = jnp.zeros_like(sq)
        row08 = jax.lax.broadcasted_iota(jnp.int32, (8, _KC), 0) == 0
        enc_q[_NQ:_MA, :] = jnp.where(row08, 1.0, 0.0).astype(jnp.bfloat16)

    @pl.when(i >= _NCHUNK)
    def _phase1():
        j = i - _NCHUNK
        q = qvm[j]
        w = wvm[j]
        one_q = jnp.ones_like(q)
        zero_q = jnp.zeros_like(q)
        one_w = jnp.ones_like(w)
        two_w = one_w + one_w
        zero_w = jnp.zeros_like(w)
        ones_q = zero_q
        for k in range(_NCH):
            cq = jnp.where(q > tq[:, k:k + 1], one_q, zero_q)
            cw = jnp.where(w > tw[:, k:k + 1],
                           two_w if k == 3 else one_w, zero_w)
            ones_q = ones_q + (cq + cq if k == 3 else cq)
            enc_q[0:_NQ, k * _CHUNK:(k + 1) * _CHUNK] = cq
            enc_w[0:_NCLS, k * _CHUNK:(k + 1) * _CHUNK] = cw
        sq[:] += jnp.sum(ones_q.astype(jnp.float32), axis=1, keepdims=True)
        acc[:] += jax.lax.dot_general(
            enc_q[:], enc_w[:],
            dimension_numbers=(((1,), (1,)), ((), ())),
            preferred_element_type=jnp.float32,
        )

    @pl.when(i == 2 * _NCHUNK - 1)
    def _fin():
        sw = acc[_NQ:_NQ + 1, 0:_NCLS]
        out_ref[:] = (_DTOT + 2.0 * acc[0:_NQ, 0:_NCLS]) - sq[:] - sw


def kernel(query, weight):
    return pl.pallas_call(
        _body,
        grid=(2 * _NCHUNK,),
        in_specs=[
            pl.BlockSpec((_NQ, _CHUNK),
                         lambda i: (0, jnp.minimum(i, _NCHUNK - 1))),
            pl.BlockSpec((_NCLS, _CHUNK),
                         lambda i: (0, jnp.minimum(i, _NCHUNK - 1))),
        ],
        out_specs=pl.BlockSpec((_NQ, _NCLS), lambda i: (0, 0)),
        out_shape=jax.ShapeDtypeStruct((_NQ, _NCLS), jnp.float32),
        scratch_shapes=[
            pltpu.VMEM((_NCHUNK, _NQ, _CHUNK), jnp.bfloat16),
            pltpu.VMEM((_NCHUNK, _NCLS, _CHUNK), jnp.bfloat16),
            pltpu.VMEM((_NQ, _CHUNK), jnp.float32),
            pltpu.VMEM((_NCLS, _CHUNK), jnp.float32),
            pltpu.VMEM((_NCLS, _CHUNK), jnp.float32),
            pltpu.VMEM((_NCLS, _CHUNK), jnp.float32),
            pltpu.VMEM((_NQ, 8), jnp.bfloat16),
            pltpu.VMEM((_NCLS, 8), jnp.bfloat16),
            pltpu.VMEM((_MA, _KC), jnp.bfloat16),
            pltpu.VMEM((_NCLS, _KC), jnp.bfloat16),
            pltpu.VMEM((_MA, _NCLS), jnp.float32),
            pltpu.VMEM((_NQ, 1), jnp.float32),
        ],
    )(query, weight)


# speedup vs baseline: 1.1076x; 1.0821x over previous
import jax
import jax.numpy as jnp
from jax.experimental import pallas as pl
from jax.experimental.pallas import tpu as pltpu

_DIM = 2048
_NCLS = 512
_NQ = 1024
_NLEV = 8
_CHUNK = 512
_NCHUNK = _DIM // _CHUNK
_DTOT = _DIM * 8
_NCH = 7
_KC = _NCH * _CHUNK
_MA = _NQ + 8


def _body(q_ref, w_ref, out_ref,
          qvm, wvm, qsq, wsq, wmn, wmx, tq, tw, enc_q, enc_w, acc, sq):
    i = pl.program_id(0)

    @pl.when(i == 0)
    def _init0():
        qsq[:] = jnp.zeros_like(qsq)
        wsq[:] = jnp.zeros_like(wsq)
        wmn[:] = jnp.full_like(wmn, jnp.inf)
        wmx[:] = jnp.full_like(wmx, -jnp.inf)

    @pl.when(i < _NCHUNK)
    def _phase0():
        q = q_ref[:]
        w = w_ref[:]
        qvm[i] = q.astype(jnp.bfloat16)
        wvm[i] = w.astype(jnp.bfloat16)
        qsq[:] += q * q
        wsq[:] += w * w
        wmn[:] = jnp.minimum(wmn[:], w)
        wmx[:] = jnp.maximum(wmx[:], w)

    @pl.when(i == _NCHUNK - 1)
    def _mk_thresholds():
        qnorm = jnp.sqrt(jnp.sum(qsq[:], axis=1, keepdims=True))
        wnorm = jnp.sqrt(jnp.sum(wsq[:], axis=1, keepdims=True))
        gmin = jnp.min(jnp.min(wmn[:], axis=1, keepdims=True) / wnorm)
        gmax = jnp.max(jnp.max(wmx[:], axis=1, keepdims=True) / wnorm)
        step = (gmax - gmin) / (_NLEV - 1)
        k = jax.lax.broadcasted_iota(jnp.int32, (1, 8), 1).astype(jnp.float32)
        mids = gmin + (k + 0.5) * step
        tq[:] = (mids * qnorm).astype(jnp.bfloat16)
        tw[:] = (mids * wnorm).astype(jnp.bfloat16)

    @pl.when(i == _NCHUNK)
    def _init1():
        acc[:] = jnp.zeros_like(acc)
        sq[:] = jnp.zeros_like(sq)
        row08 = jax.lax.broadcasted_iota(jnp.int32, (8, _KC), 0) == 0
        enc_q[_NQ:_MA, :] = jnp.where(row08, 1.0, 0.0).astype(jnp.float8_e4m3fn)

    @pl.when(i >= _NCHUNK)
    def _phase1():
        j = i - _NCHUNK
        q = qvm[j]
        w = wvm[j]
        one_q = jnp.ones_like(q)
        zero_q = jnp.zeros_like(q)
        one_w = jnp.ones_like(w)
        two_w = one_w + one_w
        zero_w = jnp.zeros_like(w)
        ones_q = zero_q
        for k in range(_NCH):
            cq = jnp.where(q > tq[:, k:k + 1], one_q, zero_q)
            cw = jnp.where(w > tw[:, k:k + 1],
                           two_w if k == 3 else one_w, zero_w)
            ones_q = ones_q + (cq + cq if k == 3 else cq)
            enc_q[0:_NQ, k * _CHUNK:(k + 1) * _CHUNK] = cq.astype(jnp.float8_e4m3fn)
            enc_w[0:_NCLS, k * _CHUNK:(k + 1) * _CHUNK] = cw.astype(jnp.float8_e4m3fn)
        sq[:] += jnp.sum(ones_q.astype(jnp.float32), axis=1, keepdims=True)
        acc[:] += jax.lax.dot_general(
            enc_q[:], enc_w[:],
            dimension_numbers=(((1,), (1,)), ((), ())),
            preferred_element_type=jnp.float32,
        )

    @pl.when(i == 2 * _NCHUNK - 1)
    def _fin():
        sw = acc[_NQ:_NQ + 1, 0:_NCLS]
        out_ref[:] = (_DTOT + 2.0 * acc[0:_NQ, 0:_NCLS]) - sq[:] - sw


def kernel(query, weight):
    return pl.pallas_call(
        _body,
        grid=(2 * _NCHUNK,),
        in_specs=[
            pl.BlockSpec((_NQ, _CHUNK),
                         lambda i: (0, jnp.minimum(i, _NCHUNK - 1))),
            pl.BlockSpec((_NCLS, _CHUNK),
                         lambda i: (0, jnp.minimum(i, _NCHUNK - 1))),
        ],
        out_specs=pl.BlockSpec((_NQ, _NCLS), lambda i: (0, 0)),
        out_shape=jax.ShapeDtypeStruct((_NQ, _NCLS), jnp.float32),
        scratch_shapes=[
            pltpu.VMEM((_NCHUNK, _NQ, _CHUNK), jnp.bfloat16),
            pltpu.VMEM((_NCHUNK, _NCLS, _CHUNK), jnp.bfloat16),
            pltpu.VMEM((_NQ, _CHUNK), jnp.float32),
            pltpu.VMEM((_NCLS, _CHUNK), jnp.float32),
            pltpu.VMEM((_NCLS, _CHUNK), jnp.float32),
            pltpu.VMEM((_NCLS, _CHUNK), jnp.float32),
            pltpu.VMEM((_NQ, 8), jnp.bfloat16),
            pltpu.VMEM((_NCLS, 8), jnp.bfloat16),
            pltpu.VMEM((_MA, _KC), jnp.float8_e4m3fn),
            pltpu.VMEM((_NCLS, _KC), jnp.float8_e4m3fn),
            pltpu.VMEM((_MA, _NCLS), jnp.float32),
            pltpu.VMEM((_NQ, 1), jnp.float32),
        ],
    )(query, weight)
